# Initial kernel scaffold; baseline (speedup 1.0000x reference)
import jax
import jax.numpy as jnp
from jax.experimental import pallas as pl


def kernel(in_feat, edge_index, graph_ids, emb, ws1, wn1, b1, al1, gm1, bt1, ws2, wn2, b2, al2, gm2, bt2, ws3, wn3, b3, al3, gm3, bt3, ws4, wn4, b4, al4, gm4, bt4):
    raise NotImplementedError("write your pallas kernel here")


# speedup vs baseline: 2.1894x; 2.1894x over previous
import functools

import jax
import jax.numpy as jnp
from jax import lax
from jax.experimental import pallas as pl
from jax.experimental.pallas import tpu as pltpu
from jax.experimental.pallas import tpu_sc as plsc

NN = 50000
EE = 800000
DD = 64
GG = 100
FN = float(NN)

NH = 25024
NPAD = 2 * NH
NHA = 12512
ACC_ROWS = 12528
EPT = 50176
E_PAD = 16 * EPT
BLK = 512
NBLK = EPT // BLK

HPT = 1568
HPAD = 32 * HPT
DEG_ROWS = 16 * 1568

BN_ROWS = 2000
NG = NN // BN_ROWS
GP = 104

_mesh = plsc.VectorSubcoreMesh(core_axis_name="c", subcore_axis_name="s")
_sc_params = pltpu.CompilerParams(use_tc_tiling_on_sc=False,
                                  internal_scratch_in_bytes=128 * 1024)


@functools.partial(
    pl.kernel,
    out_type=jax.ShapeDtypeStruct((HPAD, DD), jnp.float32),
    mesh=_mesh,
    scratch_types=[
        pltpu.VMEM((HPT,), jnp.int32),
        pltpu.VMEM((HPT, DD), jnp.float32),
        pltpu.SemaphoreType.DMA,
    ],
    compiler_params=_sc_params,
)
def _sc_embed(emb_hbm, feat_hbm, h_hbm, idx_v, rows_v, sem):
    c = lax.axis_index("c")
    s = lax.axis_index("s")
    w = s * 2 + c
    base = w * HPT
    pltpu.sync_copy(feat_hbm.at[pl.ds(base, HPT)], idx_v)
    cps = []
    for j in range(12):
        cps.append(pltpu.async_copy(
            emb_hbm.at[idx_v.at[pl.ds(j * 128, 128)]],
            rows_v.at[pl.ds(j * 128, 128)], sem))
    cps.append(pltpu.async_copy(
        emb_hbm.at[idx_v.at[pl.ds(1536, 32)]],
        rows_v.at[pl.ds(1536, 32)], sem))
    for cp in cps:
        cp.wait()
    pltpu.sync_copy(rows_v, h_hbm.at[pl.ds(base, HPT)])


@functools.partial(
    pl.kernel,
    out_type=jax.ShapeDtypeStruct((NPAD,), jnp.float32),
    mesh=_mesh,
    scratch_types=[
        pltpu.VMEM((BLK,), jnp.int32),
        pltpu.VMEM((4, 128), jnp.int32),
        pltpu.VMEM((128,), jnp.float32),
        pltpu.VMEM((128,), jnp.float32),
        pltpu.VMEM_SHARED((DEG_ROWS,), jnp.float32),
        pltpu.SemaphoreType.DMA,
    ],
    compiler_params=_sc_params,
)
def _sc_deg(dst_hbm, deg_hbm, dst_v, ldst_v, ones_v, zero_v, acc, sem):
    c = lax.axis_index("c")
    s = lax.axis_index("s")
    ones16 = jnp.ones((16,), jnp.float32)
    zero16 = jnp.zeros((16,), jnp.float32)
    for k in range(8):
        ones_v[pl.ds(k * 16, 16)] = ones16
        zero_v[pl.ds(k * 16, 16)] = zero16
    zbase = s * 1568
    zs = [pltpu.async_copy(zero_v, acc.at[pl.ds(zbase + q * 128, 128)], sem)
          for q in range(12)]
    zs.append(pltpu.async_copy(zero_v.at[pl.ds(0, 32)],
                               acc.at[pl.ds(zbase + 1536, 32)], sem))
    for cp in zs:
        cp.wait()
    plsc.subcore_barrier()

    lo = c * NH

    def body(bi, _):
        eoff = s * EPT + bi * BLK
        pltpu.sync_copy(dst_hbm.at[pl.ds(eoff, BLK)], dst_v)
        for kk in range(BLK // 16):
            d = dst_v[pl.ds(kk * 16, 16)]
            l = d - lo
            own = (l >= 0) & (l < NH)
            ldst_v[kk // 8, pl.ds((kk % 8) * 16, 16)] = jnp.where(own, l, NH)
        cps = [pltpu.async_copy(ones_v, acc.at[ldst_v.at[j]], sem, add=True)
               for j in range(4)]
        for cp in cps:
            cp.wait()
        return ()

    lax.fori_loop(0, NBLK, body, ())
    plsc.subcore_barrier()

    @pl.when(s < 8)
    def _():
        pltpu.sync_copy(acc.at[pl.ds(s * 3128, 3128)],
                        deg_hbm.at[pl.ds(c * NH + s * 3128, 3128)])


@functools.partial(
    pl.kernel,
    out_type=jax.ShapeDtypeStruct((NPAD, DD), jnp.float32),
    mesh=_mesh,
    scratch_types=[
        pltpu.VMEM((BLK,), jnp.int32),
        pltpu.VMEM((BLK,), jnp.int32),
        pltpu.VMEM((BLK + 16,), jnp.int32),
        pltpu.VMEM((BLK + 16,), jnp.int32),
        pltpu.VMEM((4, 128), jnp.int32),
        pltpu.VMEM((BLK, DD), jnp.float32),
        pltpu.VMEM((128, DD), jnp.float32),
        pltpu.VMEM_SHARED((ACC_ROWS, DD), jnp.float32),
        pltpu.SemaphoreType.DMA,
        pltpu.SemaphoreType.DMA,
    ],
    compiler_params=_sc_params,
)
def _sc_agg(p_hbm, src_hbm, dst_hbm, out_hbm,
            src_v, dst_v, csrc_v, cfl_v, cld_v, rows_v, zbuf, acc,
            sem_g, sem_s):
    c = lax.axis_index("c")
    s = lax.axis_index("s")
    zero16 = jnp.zeros((16,), jnp.float32)

    def zrow(r, _):
        for k in range(4):
            zbuf[r, pl.ds(k * 16, 16)] = zero16
        return ()

    lax.fori_loop(0, 128, zrow, ())

    def zero_acc(rows_per_tile):
        zb = s * rows_per_tile
        nfull, rem = divmod(rows_per_tile, 128)
        cps = [pltpu.async_copy(zbuf.at[pl.ds(0, 128)],
                                acc.at[pl.ds(zb + q * 128, 128)], sem_g)
               for q in range(nfull)]
        if rem:
            cps.append(pltpu.async_copy(
                zbuf.at[pl.ds(0, rem)],
                acc.at[pl.ds(zb + nfull * 128, rem)], sem_g))
        for cp in cps:
            cp.wait()

    def do_pass(base, width, dump):
        zero16i = jnp.zeros((16,), jnp.int32)
        dump16 = jnp.full((16,), dump, jnp.int32)

        def body(bi, _):
            eoff = s * EPT + bi * BLK
            pltpu.sync_copy(src_hbm.at[pl.ds(eoff, BLK)], src_v)
            pltpu.sync_copy(dst_hbm.at[pl.ds(eoff, BLK)], dst_v)
            for g in range(BLK // 16 + 1):
                csrc_v[pl.ds(g * 16, 16)] = zero16i
                cfl_v[pl.ds(g * 16, 16)] = dump16
            for kk in range(BLK // 16):
                d = dst_v[pl.ds(kk * 16, 16)]
                l = d - base
                m = (l >= 0) & (l < width)
                sv = src_v[pl.ds(kk * 16, 16)]
                csrc_v[pl.ds(kk * 16, 16)] = sv
                cfl_v[pl.ds(kk * 16, 16)] = jnp.where(m, l, dump)
            for g in range(BLK // 16):
                cld_v[g // 8, pl.ds((g % 8) * 16, 16)] = cfl_v[pl.ds(g * 16, 16)]
            def chunk_body(j, _):
                off = j * 128
                pltpu.async_copy(
                    p_hbm.at[csrc_v.at[pl.ds(off, 128)]],
                    rows_v.at[pl.ds(off, 128)], sem_g).wait()
                pltpu.sync_copy(rows_v.at[pl.ds(off, 128)],
                                acc.at[cld_v.at[j]], add=True)
                return ()

            for j in range(4):
                chunk_body(j, ())
            return ()

        lax.fori_loop(0, NBLK, body, ())

    for half in range(2):
        zero_acc(ACC_ROWS // 16)
        plsc.subcore_barrier()
        glo = (2 * half + c) * NHA
        do_pass(glo, NHA, NHA)
        plsc.subcore_barrier()
        ob = s * (NHA // 16)
        pltpu.sync_copy(acc.at[pl.ds(ob, NHA // 16)],
                        out_hbm.at[pl.ds(glo + ob, NHA // 16)])
        plsc.subcore_barrier()


def _dense_body(sums_ref, gmp_ref, btp_ref, ws_ref, wn_ref, b_ref, al_ref,
                p_ref, agg_ref, deg_ref, pout_ref, sums_out_ref):
    i = pl.program_id(0)
    mu = sums_ref[0:1, :] / FN
    var = sums_ref[1:2, :] / FN - mu * mu
    sa = gmp_ref[...] * lax.rsqrt(var + 1e-5)
    ta = btp_ref[...] - mu * sa
    st = jnp.transpose(sa)
    wsf = st * ws_ref[...]
    wnf = st * wn_ref[...]
    c0 = jnp.dot(ta, ws_ref[...], preferred_element_type=jnp.float32) + b_ref[...]
    c1 = jnp.dot(ta, wn_ref[...], preferred_element_type=jnp.float32)
    deg = deg_ref[...]
    inv = 1.0 / jnp.maximum(deg, 1.0)
    msk = (deg > 0).astype(jnp.float32)
    r = (jnp.dot(p_ref[...], wsf, preferred_element_type=jnp.float32)
         + jnp.dot(agg_ref[...] * inv, wnf, preferred_element_type=jnp.float32)
         + c0 + msk * c1)
    al = al_ref[...]
    pp = jnp.where(r >= 0, r, al * r)
    pout_ref[...] = pp
    stats = jnp.concatenate(
        [jnp.sum(pp, axis=0, keepdims=True),
         jnp.sum(pp * pp, axis=0, keepdims=True)], axis=0)

    @pl.when(i == 0)
    def _():
        sums_out_ref[...] = stats

    @pl.when(i > 0)
    def _():
        sums_out_ref[...] += stats


def _dense(sums, gmp, btp, ws, wn, b, al, p, agg, deg):
    full = lambda shp: pl.BlockSpec(shp, lambda i: (0, 0))
    return pl.pallas_call(
        _dense_body,
        grid=(NG,),
        in_specs=[
            full((2, DD)), full((1, DD)), full((1, DD)),
            full((DD, DD)), full((DD, DD)), full((1, DD)), full((1, DD)),
            pl.BlockSpec((BN_ROWS, DD), lambda i: (i, 0)),
            pl.BlockSpec((BN_ROWS, DD), lambda i: (i, 0)),
            pl.BlockSpec((BN_ROWS, 1), lambda i: (i, 0)),
        ],
        out_specs=[
            pl.BlockSpec((BN_ROWS, DD), lambda i: (i, 0)),
            full((2, DD)),
        ],
        out_shape=[
            jax.ShapeDtypeStruct((NN, DD), jnp.float32),
            jax.ShapeDtypeStruct((2, DD), jnp.float32),
        ],
    )(sums, gmp, btp, ws, wn, b, al, p, agg, deg)


def _affine(su, gm, bt):
    mu = su[0:1, :] / FN
    var = su[1:2, :] / FN - mu * mu
    sa = gm * lax.rsqrt(var + 1e-5)
    return sa, bt - mu * sa


def _readout_body(su1, gm1, bt1, su2, gm2, bt2, su3, gm3, bt3, su4, gm4, bt4,
                  ids_ref, p1, p2, p3, p4, out_ref, acc, cnt):
    i = pl.program_id(0)

    @pl.when(i == 0)
    def _():
        acc[...] = jnp.zeros_like(acc)
        cnt[...] = jnp.zeros_like(cnt)

    ids = ids_ref[...]
    oh = (lax.broadcasted_iota(jnp.int32, (BN_ROWS, GP), 1) == ids
          ).astype(jnp.float32)
    dn = (((0,), (0,)), ((), ()))
    for li, pref in enumerate((p1, p2, p3, p4)):
        m = lax.dot_general(oh, pref[...], dn, preferred_element_type=jnp.float32)
        acc[:, 64 * li:64 * (li + 1)] += m
    cnt[...] += lax.dot_general(oh, jnp.ones((BN_ROWS, 128), jnp.float32), dn,
                                preferred_element_type=jnp.float32)

    @pl.when(i == NG - 1)
    def _():
        pairs = [_affine(su1[...], gm1[...], bt1[...]),
                 _affine(su2[...], gm2[...], bt2[...]),
                 _affine(su3[...], gm3[...], bt3[...]),
                 _affine(su4[...], gm4[...], bt4[...])]
        S = jnp.concatenate([a for a, _ in pairs], axis=1)
        T = jnp.concatenate([t for _, t in pairs], axis=1)
        c1 = cnt[:, 0:1]
        mean = acc[...] / jnp.maximum(c1, 1.0)
        out_ref[...] = jnp.where(c1 > 0, mean * S + T, 0.0)


def _readout(su1, gm1, bt1, su2, gm2, bt2, su3, gm3, bt3, su4, gm4, bt4,
             ids, p1, p2, p3, p4):
    full = lambda shp: pl.BlockSpec(shp, lambda i: (0, 0))
    blk = pl.BlockSpec((BN_ROWS, DD), lambda i: (i, 0))
    return pl.pallas_call(
        _readout_body,
        grid=(NG,),
        in_specs=[full((2, DD)), full((1, DD)), full((1, DD))] * 4
        + [pl.BlockSpec((BN_ROWS, 1), lambda i: (i, 0)), blk, blk, blk, blk],
        out_specs=full((GP, 4 * DD)),
        out_shape=jax.ShapeDtypeStruct((GP, 4 * DD), jnp.float32),
        scratch_shapes=[
            pltpu.VMEM((GP, 4 * DD), jnp.float32),
            pltpu.VMEM((GP, 128), jnp.float32),
        ],
    )(su1, gm1, bt1, su2, gm2, bt2, su3, gm3, bt3, su4, gm4, bt4,
      ids, p1, p2, p3, p4)


def kernel(in_feat, edge_index, graph_ids, emb,
           ws1, wn1, b1, al1, gm1, bt1,
           ws2, wn2, b2, al2, gm2, bt2,
           ws3, wn3, b3, al3, gm3, bt3,
           ws4, wn4, b4, al4, gm4, bt4):
    src = edge_index[0]
    dst = edge_index[1]
    srcp = jnp.concatenate([src, jnp.zeros((E_PAD - EE,), src.dtype)])
    dstp = jnp.concatenate([dst, jnp.full((E_PAD - EE,), NPAD - 1, dst.dtype)])
    featp = jnp.concatenate(
        [in_feat.astype(jnp.int32), jnp.zeros((HPAD - NN,), jnp.int32)])

    h = _sc_embed(emb, featp)
    deg = _sc_deg(dstp).reshape(NPAD, 1)

    r1 = lambda v: v.reshape(1, DD)
    ones_r = jnp.ones((1, DD), jnp.float32)
    zeros_r = jnp.zeros((1, DD), jnp.float32)
    sums0 = jnp.stack([jnp.zeros((DD,), jnp.float32),
                       jnp.full((DD,), FN * (1.0 - 1e-5), jnp.float32)])

    agg1 = _sc_agg(h, srcp, dstp)
    p1, sums1 = _dense(sums0, ones_r, zeros_r, ws1, wn1, r1(b1), r1(al1),
                       h, agg1, deg)
    agg2 = _sc_agg(p1, srcp, dstp)
    p2, sums2 = _dense(sums1, r1(gm1), r1(bt1), ws2, wn2, r1(b2), r1(al2),
                       p1, agg2, deg)
    agg3 = _sc_agg(p2, srcp, dstp)
    p3, sums3 = _dense(sums2, r1(gm2), r1(bt2), ws3, wn3, r1(b3), r1(al3),
                       p2, agg3, deg)
    agg4 = _sc_agg(p3, srcp, dstp)
    p4, sums4 = _dense(sums3, r1(gm3), r1(bt3), ws4, wn4, r1(b4), r1(al4),
                       p3, agg4, deg)

    ids = graph_ids.astype(jnp.int32).reshape(NN, 1)
    out = _readout(sums1, r1(gm1), r1(bt1), sums2, r1(gm2), r1(bt2),
                   sums3, r1(gm3), r1(bt3), sums4, r1(gm4), r1(bt4),
                   ids, p1, p2, p3, p4)
    return out[:GG]
